# Initial kernel scaffold; baseline (speedup 1.0000x reference)
import jax
import jax.numpy as jnp
from jax.experimental import pallas as pl


def kernel(x, labels, centers):
    raise NotImplementedError("write your pallas kernel here")


# speedup vs baseline: 24.1869x; 24.1869x over previous
import functools

import jax
import jax.numpy as jnp
from jax import lax
from jax.experimental import pallas as pl
from jax.experimental.pallas import tpu as pltpu
from jax.experimental.pallas import tpu_sc as plsc

NUM_CLASS = 8192
DIMS = 256
NUM_HEADS = 4
HEAD_DIM = DIMS // NUM_HEADS
ALPHA = 0.999
BATCH = 16384

NC = 2
NS = 16
NW = NC * NS
LANES = 16
ROWS_PER_W = BATCH // NW
CHUNK = 128
NCHUNK = ROWS_PER_W // CHUNK
GROUPS = DIMS // LANES


def _ema_body(x_hbm, lab_hbm, cen_hbm, out_hbm, loss_hbm,
              idx_v, x_v, c_v, acc_v, sem):
    wid = lax.axis_index("s") * NC + lax.axis_index("c")
    base = wid * ROWS_PER_W
    acc_v[...] = jnp.zeros((LANES,), jnp.float32)
    for k in range(NCHUNK):
        row0 = base + k * CHUNK
        pltpu.sync_copy(lab_hbm.at[pl.ds(row0, CHUNK)], idx_v)
        pltpu.sync_copy(x_hbm.at[pl.ds(row0, CHUNK)], x_v)
        pltpu.async_copy(cen_hbm.at[idx_v], c_v, sem).wait()

        def row_body(i, acc):
            for j in range(GROUPS):
                xv = x_v[i, pl.ds(j * LANES, LANES)]
                cv = c_v[i, pl.ds(j * LANES, LANES)]
                d = xv - cv
                acc = acc + d * d
                c_v[i, pl.ds(j * LANES, LANES)] = cv + (1.0 - ALPHA) * d
            return acc

        acc_v[...] = lax.fori_loop(0, CHUNK, row_body, acc_v[...])
        pltpu.async_copy(c_v, out_hbm.at[idx_v], sem).wait()
    pltpu.sync_copy(acc_v, loss_hbm.at[wid])


_ema_sc = functools.partial(
    pl.kernel,
    out_type=jax.ShapeDtypeStruct((NW, LANES), jnp.float32),
    mesh=plsc.VectorSubcoreMesh(core_axis_name="c", subcore_axis_name="s",
                                num_cores=NC, num_subcores=NS),
    scratch_types=[
        pltpu.VMEM((CHUNK,), jnp.int32),
        pltpu.VMEM((CHUNK, DIMS), jnp.float32),
        pltpu.VMEM((CHUNK, DIMS), jnp.float32),
        pltpu.VMEM((LANES,), jnp.float32),
        pltpu.SemaphoreType.DMA,
    ],
)(_ema_body)


def kernel(x, labels, centers):
    cen2 = centers.reshape(NUM_CLASS, DIMS)
    out_ref = jax.new_ref(cen2)
    loss_parts = _ema_sc(x, labels, cen2, out_ref)
    new_centers = out_ref[...].reshape(NUM_CLASS, NUM_HEADS, HEAD_DIM)
    loss = jnp.sum(loss_parts) / (BATCH * HEAD_DIM)
    return loss, new_centers
